# baseline (speedup 1.0000x reference)
import functools

import jax
import jax.numpy as jnp
from jax.experimental import pallas as pl
from jax.experimental.pallas import tpu as pltpu

_K = 64
_T = 64


def _encode_body(x_ref, w_ref, benc_ref, bdec_ref, z_ref, l0_ref, *, k):
    i = pl.program_id(0)
    xc = x_ref[...] - bdec_ref[...]
    pre = jnp.dot(xc, w_ref[...], preferred_element_type=jnp.float32)
    pre = pre + benc_ref[...]

    d_hid = pre.shape[1]
    ones = jnp.ones((d_hid, 8), jnp.float32)

    def count_rows(mask):
        mf = jnp.where(mask, 1.0, 0.0)
        c = jnp.dot(mf, ones, preferred_element_type=jnp.float32)[:, :1]
        return c.astype(jnp.int32)

    s = jax.lax.bitcast_convert_type(pre, jnp.int32)
    key = jnp.where(s < 0, s ^ jnp.int32(0x7FFFFFFF), s)

    cnt_nonneg = count_rows(key >= 0)
    use_neg = cnt_nonneg < k
    vdom = key & jnp.int32(0x7FFFFFFF)
    include = (key < 0) == use_neg
    arr = jnp.where(include, vdom, jnp.int32(-1))
    kp = jnp.where(use_neg, k - cnt_nonneg, k)

    t = jnp.zeros(arr.shape[:1] + (1,), jnp.int32)
    for b in range(30, -1, -1):
        cand = t | jnp.int32(1 << b)
        cnt = count_rows(arr >= cand)
        t = jnp.where(cnt >= kp, cand, t)

    t_key = jnp.where(use_neg, t + jnp.int32(-2147483648), t)
    sbits = jnp.where(t_key < 0, t_key ^ jnp.int32(0x7FFFFFFF), t_key)
    thr = jax.lax.bitcast_convert_type(sbits, jnp.float32)

    zb = jnp.where(pre >= thr, jnp.maximum(pre, 0.0), 0.0)
    z_ref[...] = zb

    @pl.when(i == 0)
    def _():
        l0_ref[0, 0] = 0.0

    l0_ref[0, 0] += jnp.sum((zb > 0.0).astype(jnp.float32))


def _decode_body(z_ref, w_ref, bdec_ref, x_ref, xhat_ref, sq_ref):
    i = pl.program_id(0)
    xh = jnp.dot(z_ref[...], w_ref[...], preferred_element_type=jnp.float32)
    xh = xh + bdec_ref[...]
    xhat_ref[...] = xh
    d = xh - x_ref[...]

    @pl.when(i == 0)
    def _():
        sq_ref[0, 0] = 0.0

    sq_ref[0, 0] += jnp.sum(d * d)


def kernel(x, W_enc, b_enc, W_dec, b_dec):
    n_tok, d_in = x.shape
    d_hid = W_enc.shape[0]
    t = min(_T, n_tok)
    nt = n_tok // t

    w_enc_t = W_enc.T
    w_dec_t = W_dec.T
    benc2 = b_enc.reshape(1, d_hid)
    bdec2 = b_dec.reshape(1, d_in)

    cparams = pltpu.CompilerParams(vmem_limit_bytes=100 * 1024 * 1024)

    z, l0_sum = pl.pallas_call(
        functools.partial(_encode_body, k=_K),
        grid=(nt,),
        compiler_params=cparams,
        in_specs=[
            pl.BlockSpec((t, d_in), lambda i: (i, 0)),
            pl.BlockSpec((d_in, d_hid), lambda i: (0, 0)),
            pl.BlockSpec((1, d_hid), lambda i: (0, 0)),
            pl.BlockSpec((1, d_in), lambda i: (0, 0)),
        ],
        out_specs=[
            pl.BlockSpec((t, d_hid), lambda i: (i, 0)),
            pl.BlockSpec((1, 1), lambda i: (0, 0), memory_space=pltpu.SMEM),
        ],
        out_shape=[
            jax.ShapeDtypeStruct((n_tok, d_hid), jnp.float32),
            jax.ShapeDtypeStruct((1, 1), jnp.float32),
        ],
    )(x, w_enc_t, benc2, bdec2)

    x_hat, sq_sum = pl.pallas_call(
        _decode_body,
        grid=(nt,),
        compiler_params=cparams,
        in_specs=[
            pl.BlockSpec((t, d_hid), lambda i: (i, 0)),
            pl.BlockSpec((d_hid, d_in), lambda i: (0, 0)),
            pl.BlockSpec((1, d_in), lambda i: (0, 0)),
            pl.BlockSpec((t, d_in), lambda i: (i, 0)),
        ],
        out_specs=[
            pl.BlockSpec((t, d_in), lambda i: (i, 0)),
            pl.BlockSpec((1, 1), lambda i: (0, 0), memory_space=pltpu.SMEM),
        ],
        out_shape=[
            jax.ShapeDtypeStruct((n_tok, d_in), jnp.float32),
            jax.ShapeDtypeStruct((1, 1), jnp.float32),
        ],
    )(z, w_dec_t, bdec2, x)

    recon_loss = (sq_sum / (n_tok * d_in))[0, 0]
    l0 = (l0_sum / n_tok)[0, 0]
    loss = recon_loss
    return (x_hat, z, loss, recon_loss, l0)


# speedup vs baseline: 1.4314x; 1.4314x over previous
import functools

import jax
import jax.numpy as jnp
from jax.experimental import pallas as pl
from jax.experimental.pallas import tpu as pltpu

_K = 64
_TE = 64
_TD = 128


def _encode_body(x_ref, w_ref, benc_ref, bdec_ref, z_ref, l0_ref, pre_ref,
                 *, k):
    i = pl.program_id(0)

    pre = pre_ref[...]

    s = jax.lax.bitcast_convert_type(pre, jnp.int32)
    key = jnp.where(s < 0, s ^ jnp.int32(0x7FFFFFFF), s)

    cnt_nonneg = jnp.sum((key >= 0).astype(jnp.int32), axis=1, keepdims=True)
    use_neg = cnt_nonneg < k
    vdom = key & jnp.int32(0x7FFFFFFF)
    include = (key < 0) == use_neg
    arr = jnp.where(include, vdom, jnp.int32(-1))
    kp = jnp.where(use_neg, k - cnt_nonneg, k)

    t = jnp.zeros(arr.shape[:1] + (1,), jnp.int32)
    for b in range(30, -1, -1):
        cand = t | jnp.int32(1 << b)
        cnt = jnp.sum((arr >= cand).astype(jnp.int32), axis=1, keepdims=True)
        t = jnp.where(cnt >= kp, cand, t)

    t_key = jnp.where(use_neg, t + jnp.int32(-2147483648), t)
    sbits = jnp.where(t_key < 0, t_key ^ jnp.int32(0x7FFFFFFF), t_key)
    thr = jax.lax.bitcast_convert_type(sbits, jnp.float32)

    zb = jnp.where(pre >= thr, jnp.maximum(pre, 0.0), 0.0)
    z_ref[...] = zb

    xc = x_ref[...] - bdec_ref[...]
    mm = jnp.dot(xc, w_ref[...], preferred_element_type=jnp.float32)
    pre_ref[...] = mm + benc_ref[...]

    @pl.when(i == 0)
    def _():
        l0_ref[0, 0] = 0.0

    @pl.when(i > 0)
    def _():
        l0_ref[0, 0] += jnp.sum((zb > 0.0).astype(jnp.float32))


def _decode_body(z_ref, w_ref, bdec_ref, x_ref, xhat_ref, sq_ref):
    i = pl.program_id(0)
    xh = jnp.dot(z_ref[...], w_ref[...], preferred_element_type=jnp.float32)
    xh = xh + bdec_ref[...]
    xhat_ref[...] = xh
    d = xh - x_ref[...]

    @pl.when(i == 0)
    def _():
        sq_ref[0, 0] = 0.0

    sq_ref[0, 0] += jnp.sum(d * d)


def kernel(x, W_enc, b_enc, W_dec, b_dec):
    n_tok, d_in = x.shape
    d_hid = W_enc.shape[0]
    t = min(_TE, n_tok)
    nt = n_tok // t
    td = min(_TD, n_tok)
    ntd = n_tok // td

    w_enc_t = W_enc.T
    w_dec_t = W_dec.T
    benc2 = b_enc.reshape(1, d_hid)
    bdec2 = b_dec.reshape(1, d_in)

    cparams = pltpu.CompilerParams(vmem_limit_bytes=100 * 1024 * 1024)

    last = nt - 1
    z, l0_sum = pl.pallas_call(
        functools.partial(_encode_body, k=_K),
        grid=(nt + 1,),
        compiler_params=cparams,
        in_specs=[
            pl.BlockSpec((t, d_in), lambda i: (jnp.minimum(i, last), 0)),
            pl.BlockSpec((d_in, d_hid), lambda i: (0, 0)),
            pl.BlockSpec((1, d_hid), lambda i: (0, 0)),
            pl.BlockSpec((1, d_in), lambda i: (0, 0)),
        ],
        out_specs=[
            pl.BlockSpec((t, d_hid), lambda i: (jnp.maximum(i - 1, 0), 0)),
            pl.BlockSpec((1, 1), lambda i: (0, 0), memory_space=pltpu.SMEM),
        ],
        out_shape=[
            jax.ShapeDtypeStruct((n_tok, d_hid), jnp.float32),
            jax.ShapeDtypeStruct((1, 1), jnp.float32),
        ],
        scratch_shapes=[pltpu.VMEM((t, d_hid), jnp.float32)],
    )(x, w_enc_t, benc2, bdec2)

    x_hat, sq_sum = pl.pallas_call(
        _decode_body,
        grid=(ntd,),
        compiler_params=cparams,
        in_specs=[
            pl.BlockSpec((td, d_hid), lambda i: (i, 0)),
            pl.BlockSpec((d_hid, d_in), lambda i: (0, 0)),
            pl.BlockSpec((1, d_in), lambda i: (0, 0)),
            pl.BlockSpec((td, d_in), lambda i: (i, 0)),
        ],
        out_specs=[
            pl.BlockSpec((td, d_in), lambda i: (i, 0)),
            pl.BlockSpec((1, 1), lambda i: (0, 0), memory_space=pltpu.SMEM),
        ],
        out_shape=[
            jax.ShapeDtypeStruct((n_tok, d_in), jnp.float32),
            jax.ShapeDtypeStruct((1, 1), jnp.float32),
        ],
    )(z, w_dec_t, bdec2, x)

    recon_loss = (sq_sum / (n_tok * d_in))[0, 0]
    l0 = (l0_sum / n_tok)[0, 0]
    loss = recon_loss
    return (x_hat, z, loss, recon_loss, l0)


# speedup vs baseline: 1.4917x; 1.0421x over previous
import functools

import jax
import jax.numpy as jnp
from jax.experimental import pallas as pl
from jax.experimental.pallas import tpu as pltpu

_K = 64
_T = 128


def _encode_body(x_ref, w_ref, benc_ref, bdec_ref, z_ref, l0_ref, *, k):
    i = pl.program_id(0)
    xc = x_ref[...] - bdec_ref[...]
    pre = jax.lax.dot_general(
        xc, w_ref[...], (((1,), (1,)), ((), ())),
        preferred_element_type=jnp.float32)
    pre = pre + benc_ref[...]

    s = jax.lax.bitcast_convert_type(pre, jnp.int32)
    key = jnp.where(s < 0, s ^ jnp.int32(0x7FFFFFFF), s)

    cnt_nonneg = jnp.sum((key >= 0).astype(jnp.int32), axis=1, keepdims=True)
    use_neg = cnt_nonneg < k
    vdom = key & jnp.int32(0x7FFFFFFF)
    include = (key < 0) == use_neg
    arr = jnp.where(include, vdom, jnp.int32(-1))
    kp = jnp.where(use_neg, k - cnt_nonneg, k)

    t = jnp.zeros(arr.shape[:1] + (1,), jnp.int32)
    for b in range(30, -1, -1):
        cand = t | jnp.int32(1 << b)
        cnt = jnp.sum((arr >= cand).astype(jnp.int32), axis=1, keepdims=True)
        t = jnp.where(cnt >= kp, cand, t)

    t_key = jnp.where(use_neg, t + jnp.int32(-2147483648), t)
    sbits = jnp.where(t_key < 0, t_key ^ jnp.int32(0x7FFFFFFF), t_key)
    thr = jax.lax.bitcast_convert_type(sbits, jnp.float32)

    zb = jnp.where(pre >= thr, jnp.maximum(pre, 0.0), 0.0)
    z_ref[...] = zb

    @pl.when(i == 0)
    def _():
        l0_ref[0, 0] = 0.0

    l0_ref[0, 0] += jnp.sum((zb > 0.0).astype(jnp.float32))


def _decode_body(z_ref, w_ref, bdec_ref, x_ref, xhat_ref, sq_ref):
    i = pl.program_id(0)
    xh = jax.lax.dot_general(
        z_ref[...], w_ref[...], (((1,), (1,)), ((), ())),
        preferred_element_type=jnp.float32)
    xh = xh + bdec_ref[...]
    xhat_ref[...] = xh
    d = xh - x_ref[...]

    @pl.when(i == 0)
    def _():
        sq_ref[0, 0] = 0.0

    sq_ref[0, 0] += jnp.sum(d * d)


def kernel(x, W_enc, b_enc, W_dec, b_dec):
    n_tok, d_in = x.shape
    d_hid = W_enc.shape[0]
    t = min(_T, n_tok)
    nt = n_tok // t

    benc2 = b_enc.reshape(1, d_hid)
    bdec2 = b_dec.reshape(1, d_in)

    cparams = pltpu.CompilerParams(vmem_limit_bytes=100 * 1024 * 1024)

    z, l0_sum = pl.pallas_call(
        functools.partial(_encode_body, k=_K),
        grid=(nt,),
        compiler_params=cparams,
        in_specs=[
            pl.BlockSpec((t, d_in), lambda i: (i, 0)),
            pl.BlockSpec((d_hid, d_in), lambda i: (0, 0)),
            pl.BlockSpec((1, d_hid), lambda i: (0, 0)),
            pl.BlockSpec((1, d_in), lambda i: (0, 0)),
        ],
        out_specs=[
            pl.BlockSpec((t, d_hid), lambda i: (i, 0)),
            pl.BlockSpec((1, 1), lambda i: (0, 0), memory_space=pltpu.SMEM),
        ],
        out_shape=[
            jax.ShapeDtypeStruct((n_tok, d_hid), jnp.float32),
            jax.ShapeDtypeStruct((1, 1), jnp.float32),
        ],
    )(x, W_enc, benc2, bdec2)

    x_hat, sq_sum = pl.pallas_call(
        _decode_body,
        grid=(nt,),
        compiler_params=cparams,
        in_specs=[
            pl.BlockSpec((t, d_hid), lambda i: (i, 0)),
            pl.BlockSpec((d_in, d_hid), lambda i: (0, 0)),
            pl.BlockSpec((1, d_in), lambda i: (0, 0)),
            pl.BlockSpec((t, d_in), lambda i: (i, 0)),
        ],
        out_specs=[
            pl.BlockSpec((t, d_in), lambda i: (i, 0)),
            pl.BlockSpec((1, 1), lambda i: (0, 0), memory_space=pltpu.SMEM),
        ],
        out_shape=[
            jax.ShapeDtypeStruct((n_tok, d_in), jnp.float32),
            jax.ShapeDtypeStruct((1, 1), jnp.float32),
        ],
    )(z, W_dec, bdec2, x)

    recon_loss = (sq_sum / (n_tok * d_in))[0, 0]
    l0 = (l0_sum / n_tok)[0, 0]
    loss = recon_loss
    return (x_hat, z, loss, recon_loss, l0)


# speedup vs baseline: 1.7185x; 1.1520x over previous
import functools

import jax
import jax.numpy as jnp
from jax.experimental import pallas as pl
from jax.experimental.pallas import tpu as pltpu

_K = 64
_T = 128


def _encode_body(x_ref, w_ref, benc_ref, bdec_ref, z_ref, l0_ref, *, k):
    i = pl.program_id(0)
    xc = x_ref[...] - bdec_ref[...]
    pre = jnp.dot(xc, w_ref[...], preferred_element_type=jnp.float32)
    pre = pre + benc_ref[...]

    s = jax.lax.bitcast_convert_type(pre, jnp.int32)
    key = jnp.where(s < 0, s ^ jnp.int32(0x7FFFFFFF), s)

    cnt_nonneg = jnp.sum((key >= 0).astype(jnp.int32), axis=1, keepdims=True)
    use_neg = cnt_nonneg < k
    vdom = key & jnp.int32(0x7FFFFFFF)
    include = (key < 0) == use_neg
    arr = jnp.where(include, vdom, jnp.int32(-1))
    kp = jnp.where(use_neg, k - cnt_nonneg, k)

    t = jnp.zeros(arr.shape[:1] + (1,), jnp.int32)
    for b in range(30, -1, -1):
        cand = t | jnp.int32(1 << b)
        cnt = jnp.sum((arr >= cand).astype(jnp.int32), axis=1, keepdims=True)
        t = jnp.where(cnt >= kp, cand, t)

    t_key = jnp.where(use_neg, t + jnp.int32(-2147483648), t)
    sbits = jnp.where(t_key < 0, t_key ^ jnp.int32(0x7FFFFFFF), t_key)
    thr = jax.lax.bitcast_convert_type(sbits, jnp.float32)

    zb = jnp.where(pre >= thr, jnp.maximum(pre, 0.0), 0.0)
    z_ref[...] = zb

    @pl.when(i == 0)
    def _():
        l0_ref[0, 0] = 0.0

    l0_ref[0, 0] += jnp.sum((zb > 0.0).astype(jnp.float32))


def _decode_body(z_ref, w_ref, bdec_ref, x_ref, xhat_ref, sq_ref):
    i = pl.program_id(0)
    xh = jnp.dot(z_ref[...], w_ref[...], preferred_element_type=jnp.float32)
    xh = xh + bdec_ref[...]
    xhat_ref[...] = xh
    d = xh - x_ref[...]

    @pl.when(i == 0)
    def _():
        sq_ref[0, 0] = 0.0

    sq_ref[0, 0] += jnp.sum(d * d)


def kernel(x, W_enc, b_enc, W_dec, b_dec):
    n_tok, d_in = x.shape
    d_hid = W_enc.shape[0]
    t = min(_T, n_tok)
    nt = n_tok // t

    w_enc_t = W_enc.T
    w_dec_t = W_dec.T
    benc2 = b_enc.reshape(1, d_hid)
    bdec2 = b_dec.reshape(1, d_in)

    cparams = pltpu.CompilerParams(vmem_limit_bytes=100 * 1024 * 1024)

    z, l0_sum = pl.pallas_call(
        functools.partial(_encode_body, k=_K),
        grid=(nt,),
        compiler_params=cparams,
        in_specs=[
            pl.BlockSpec((t, d_in), lambda i: (i, 0)),
            pl.BlockSpec((d_in, d_hid), lambda i: (0, 0)),
            pl.BlockSpec((1, d_hid), lambda i: (0, 0)),
            pl.BlockSpec((1, d_in), lambda i: (0, 0)),
        ],
        out_specs=[
            pl.BlockSpec((t, d_hid), lambda i: (i, 0)),
            pl.BlockSpec((1, 1), lambda i: (0, 0), memory_space=pltpu.SMEM),
        ],
        out_shape=[
            jax.ShapeDtypeStruct((n_tok, d_hid), jnp.float32),
            jax.ShapeDtypeStruct((1, 1), jnp.float32),
        ],
    )(x, w_enc_t, benc2, bdec2)

    x_hat, sq_sum = pl.pallas_call(
        _decode_body,
        grid=(nt,),
        compiler_params=cparams,
        in_specs=[
            pl.BlockSpec((t, d_hid), lambda i: (i, 0)),
            pl.BlockSpec((d_hid, d_in), lambda i: (0, 0)),
            pl.BlockSpec((1, d_in), lambda i: (0, 0)),
            pl.BlockSpec((t, d_in), lambda i: (i, 0)),
        ],
        out_specs=[
            pl.BlockSpec((t, d_in), lambda i: (i, 0)),
            pl.BlockSpec((1, 1), lambda i: (0, 0), memory_space=pltpu.SMEM),
        ],
        out_shape=[
            jax.ShapeDtypeStruct((n_tok, d_in), jnp.float32),
            jax.ShapeDtypeStruct((1, 1), jnp.float32),
        ],
    )(z, w_dec_t, bdec2, x)

    recon_loss = (sq_sum / (n_tok * d_in))[0, 0]
    l0 = (l0_sum / n_tok)[0, 0]
    loss = recon_loss
    return (x_hat, z, loss, recon_loss, l0)
